# baseline (speedup 1.0000x reference)
import functools

import jax
import jax.numpy as jnp
from jax import lax
from jax.experimental import pallas as pl
from jax.experimental.pallas import tpu as pltpu
from jax.experimental.pallas import tpu_sc as plsc

BATCH = 16384
EMBED_DIM = 64
NUM_ENT = 1000000
MAIN_ENT = 999424
SLICE = 249856
TAIL_PAD = 640
REL_PAD = 1024
REL_OFF = MAIN_ENT + TAIL_PAD
SPCOL_SIZE = REL_OFF + REL_PAD
NUM_REL = 1000
NUM_CORES = 2
NUM_SUBCORES = 16
ROWS_PER_TILE = BATCH // NUM_SUBCORES
DIMS_PER_CORE = EMBED_DIM // NUM_CORES
LANES = 16
IDX_CHUNK = 128
NUM_IDX_CHUNKS = ROWS_PER_TILE // IDX_CHUNK
GROUPS_PER_CHUNK = IDX_CHUNK // LANES


def _sc_partials_body(hid_hbm, tid_hbm, rx_hbm, entT_hbm, tailT_hbm,
                      relT_hbm, out_hbm, hid_v, tid_v, rx_v,
                      hbuf, tbuf, rbuf, scores_v, spcol, sem_s, sem_g):
    cid = lax.axis_index("c")
    sid = lax.axis_index("s")

    def stage_refs(j, t):
        if t < 4:
            sl = pl.ds(t * SLICE, SLICE)
            return entT_hbm.at[j, sl], spcol.at[sl]
        if t == 4:
            return tailT_hbm.at[j], spcol.at[pl.ds(MAIN_ENT, TAIL_PAD)]
        return relT_hbm.at[j], spcol.at[pl.ds(REL_OFF, REL_PAD)]

    def issue(j):
        for t in range(6):
            @pl.when(sid == t)
            def _(t=t):
                src, dst = stage_refs(j, t)
                pltpu.async_copy(src, dst, sem_s)

    def drain(j):
        for t in range(6):
            @pl.when(sid == t)
            def _(t=t):
                src, dst = stage_refs(j, t)
                pltpu.make_async_copy(src, dst, sem_s).wait()

    pltpu.sync_copy(hid_hbm.at[sid], hid_v)
    pltpu.sync_copy(tid_hbm.at[sid], tid_v)
    pltpu.sync_copy(rx_hbm.at[sid], rx_v)

    def zinit(g, carry):
        scores_v[pl.ds(g * LANES, LANES)] = jnp.zeros((LANES,), jnp.float32)
        return carry
    lax.fori_loop(0, ROWS_PER_TILE // LANES, zinit, 0)

    j0 = cid * DIMS_PER_CORE
    issue(j0)

    def dim(d, carry):
        j = j0 + d

        drain(j)
        plsc.subcore_barrier()

        ch = pltpu.async_copy(spcol.at[hid_v], hbuf, sem_g)
        ct = pltpu.async_copy(spcol.at[tid_v], tbuf, sem_g)
        cr = pltpu.async_copy(spcol.at[rx_v], rbuf, sem_g)
        ch.wait()
        ct.wait()
        cr.wait()

        plsc.subcore_barrier()

        @pl.when(d < DIMS_PER_CORE - 1)
        def _prefetch():
            issue(j + 1)

        for g in range(ROWS_PER_TILE // LANES):
            sl = pl.ds(g * LANES, LANES)
            scores_v[sl] = scores_v[sl] + hbuf[sl] * rbuf[sl] * tbuf[sl]
        return carry

    lax.fori_loop(0, DIMS_PER_CORE, dim, 0)

    pltpu.sync_copy(scores_v, out_hbm.at[cid, sid])


_sc_partials = functools.partial(
    pl.kernel,
    mesh=plsc.VectorSubcoreMesh(core_axis_name="c", subcore_axis_name="s"),
    out_type=jax.ShapeDtypeStruct((NUM_CORES, NUM_SUBCORES, ROWS_PER_TILE),
                                  jnp.float32),
    scratch_types=[
        pltpu.VMEM((ROWS_PER_TILE,), jnp.int32),
        pltpu.VMEM((ROWS_PER_TILE,), jnp.int32),
        pltpu.VMEM((ROWS_PER_TILE,), jnp.int32),
        pltpu.VMEM((ROWS_PER_TILE,), jnp.float32),
        pltpu.VMEM((ROWS_PER_TILE,), jnp.float32),
        pltpu.VMEM((ROWS_PER_TILE,), jnp.float32),
        pltpu.VMEM((ROWS_PER_TILE,), jnp.float32),
        pltpu.VMEM_SHARED((SPCOL_SIZE,), jnp.float32),
        pltpu.SemaphoreType.DMA,
        pltpu.SemaphoreType.DMA,
    ],
    compiler_params=pltpu.CompilerParams(needs_layout_passes=False),
)(_sc_partials_body)


def _softmax_body(x_ref, o_ref):
    scores = x_ref[0] + x_ref[1]
    m = jnp.max(scores)
    e = jnp.exp(scores - m)
    o_ref[...] = e * (1.0 / jnp.sum(e))


_softmax = pl.pallas_call(
    _softmax_body,
    out_shape=jax.ShapeDtypeStruct((128, 128), jnp.float32),
)


def _tiles(x):
    return x.reshape(NUM_SUBCORES, ROWS_PER_TILE)


def kernel(head_ids, rel_ids, tail_ids, entity_embeddings, relation_embeddings):
    hid = head_ids.astype(jnp.int32)
    rid = rel_ids.astype(jnp.int32)
    tid = tail_ids.astype(jnp.int32)
    entT = entity_embeddings.T
    tailT = jnp.pad(entT[:, MAIN_ENT:],
                    ((0, 0), (0, TAIL_PAD - (NUM_ENT - MAIN_ENT))))
    relT = jnp.pad(relation_embeddings.T, ((0, 0), (0, REL_PAD - NUM_REL)))
    partials = _sc_partials(_tiles(hid), _tiles(tid), _tiles(rid + REL_OFF),
                            entT, tailT, relT)
    return _softmax(partials.reshape(2, 128, 128)).reshape(BATCH)


# speedup vs baseline: 1.0145x; 1.0145x over previous
import functools

import jax
import jax.numpy as jnp
from jax import lax
from jax.experimental import pallas as pl
from jax.experimental.pallas import tpu as pltpu
from jax.experimental.pallas import tpu_sc as plsc

BATCH = 16384
EMBED_DIM = 64
NUM_ENT = 1000000
MAIN_ENT = 999424
SLICE = 249856
TAIL_PAD = 640
REL_PAD = 1024
REL_OFF = MAIN_ENT + TAIL_PAD
SPCOL_SIZE = REL_OFF + REL_PAD
NUM_REL = 1000
NUM_CORES = 2
NUM_SUBCORES = 16
ROWS_PER_TILE = BATCH // NUM_SUBCORES
DIMS_PER_CORE = EMBED_DIM // NUM_CORES
LANES = 16
IDX_CHUNK = 128
NUM_IDX_CHUNKS = ROWS_PER_TILE // IDX_CHUNK
GROUPS_PER_CHUNK = IDX_CHUNK // LANES


def _sc_partials_body(hid_hbm, tid_hbm, rx_hbm, entT_hbm, tailT_hbm,
                      relT_hbm, out_hbm, hid_v, tid_v, rx_v,
                      hbuf, tbuf, rbuf, scores_v, spcol, sem_s, sem_g):
    cid = lax.axis_index("c")
    sid = lax.axis_index("s")

    def stage_refs(j, t):
        if t < 4:
            sl = pl.ds(t * SLICE, SLICE)
            return entT_hbm.at[j, sl], spcol.at[sl]
        if t == 4:
            return tailT_hbm.at[j], spcol.at[pl.ds(MAIN_ENT, TAIL_PAD)]
        return relT_hbm.at[j], spcol.at[pl.ds(REL_OFF, REL_PAD)]

    def issue(j):
        for t in range(6):
            @pl.when(sid == t)
            def _(t=t):
                src, dst = stage_refs(j, t)
                pltpu.async_copy(src, dst, sem_s)

    def drain(j):
        for t in range(6):
            @pl.when(sid == t)
            def _(t=t):
                src, dst = stage_refs(j, t)
                pltpu.make_async_copy(src, dst, sem_s).wait()

    pltpu.sync_copy(hid_hbm.at[sid], hid_v)
    pltpu.sync_copy(tid_hbm.at[sid], tid_v)
    pltpu.sync_copy(rx_hbm.at[sid], rx_v)

    def zinit(g, carry):
        scores_v[pl.ds(g * LANES, LANES)] = jnp.zeros((LANES,), jnp.float32)
        return carry
    lax.fori_loop(0, ROWS_PER_TILE // LANES, zinit, 0)

    j0 = cid * DIMS_PER_CORE
    issue(j0)

    def dim(d, carry):
        j = j0 + d

        drain(j)
        plsc.subcore_barrier()

        copies = []
        for k in range(NUM_IDX_CHUNKS):
            copies.append(pltpu.async_copy(spcol.at[hid_v.at[k]],
                                           hbuf.at[k], sem_g))
            copies.append(pltpu.async_copy(spcol.at[tid_v.at[k]],
                                           tbuf.at[k], sem_g))
            copies.append(pltpu.async_copy(spcol.at[rx_v.at[k]],
                                           rbuf.at[k], sem_g))
        for c in copies:
            c.wait()

        plsc.subcore_barrier()

        @pl.when(d < DIMS_PER_CORE - 1)
        def _prefetch():
            issue(j + 1)

        for k in range(NUM_IDX_CHUNKS):
            for g in range(GROUPS_PER_CHUNK):
                sl = pl.ds(g * LANES, LANES)
                row0 = k * IDX_CHUNK + g * LANES
                scores_v[pl.ds(row0, LANES)] = (
                    scores_v[pl.ds(row0, LANES)]
                    + hbuf[k, sl] * rbuf[k, sl] * tbuf[k, sl])
        return carry

    lax.fori_loop(0, DIMS_PER_CORE, dim, 0)

    pltpu.sync_copy(scores_v, out_hbm.at[cid, sid])


_sc_partials = functools.partial(
    pl.kernel,
    mesh=plsc.VectorSubcoreMesh(core_axis_name="c", subcore_axis_name="s"),
    out_type=jax.ShapeDtypeStruct((NUM_CORES, NUM_SUBCORES, ROWS_PER_TILE),
                                  jnp.float32),
    scratch_types=[
        pltpu.VMEM((NUM_IDX_CHUNKS, IDX_CHUNK), jnp.int32),
        pltpu.VMEM((NUM_IDX_CHUNKS, IDX_CHUNK), jnp.int32),
        pltpu.VMEM((NUM_IDX_CHUNKS, IDX_CHUNK), jnp.int32),
        pltpu.VMEM((NUM_IDX_CHUNKS, IDX_CHUNK), jnp.float32),
        pltpu.VMEM((NUM_IDX_CHUNKS, IDX_CHUNK), jnp.float32),
        pltpu.VMEM((NUM_IDX_CHUNKS, IDX_CHUNK), jnp.float32),
        pltpu.VMEM((ROWS_PER_TILE,), jnp.float32),
        pltpu.VMEM_SHARED((SPCOL_SIZE,), jnp.float32),
        pltpu.SemaphoreType.DMA,
        pltpu.SemaphoreType.DMA,
    ],
    compiler_params=pltpu.CompilerParams(needs_layout_passes=False),
)(_sc_partials_body)


def _softmax_body(x_ref, o_ref):
    scores = x_ref[0] + x_ref[1]
    m = jnp.max(scores)
    e = jnp.exp(scores - m)
    o_ref[...] = e * (1.0 / jnp.sum(e))


_softmax = pl.pallas_call(
    _softmax_body,
    out_shape=jax.ShapeDtypeStruct((128, 128), jnp.float32),
)


def _tiles(x):
    return x.reshape(NUM_SUBCORES, NUM_IDX_CHUNKS, IDX_CHUNK)


def kernel(head_ids, rel_ids, tail_ids, entity_embeddings, relation_embeddings):
    hid = head_ids.astype(jnp.int32)
    rid = rel_ids.astype(jnp.int32)
    tid = tail_ids.astype(jnp.int32)
    entT = entity_embeddings.T
    tailT = jnp.pad(entT[:, MAIN_ENT:],
                    ((0, 0), (0, TAIL_PAD - (NUM_ENT - MAIN_ENT))))
    relT = jnp.pad(relation_embeddings.T, ((0, 0), (0, REL_PAD - NUM_REL)))
    partials = _sc_partials(_tiles(hid), _tiles(tid), _tiles(rid + REL_OFF),
                            entT, tailT, relT)
    return _softmax(partials.reshape(2, 128, 128)).reshape(BATCH)
